# baseline (speedup 1.0000x reference)
import functools

import jax
import jax.numpy as jnp
from jax import lax
from jax.experimental import pallas as pl
from jax.experimental.pallas import tpu as pltpu
from jax.experimental.pallas import tpu_sc as plsc

N = 100000
C = 40
G = 16
E = 1600000
L1 = 10
ALPHA1 = 0.8
L2 = 10
ALPHA2 = 0.8

NC = 2
NS = 16
CH = 128

BLK = 512
NBLOCK = E // BLK
NP = 97

ROWS_A = 6256
ROWS_B = N - 15 * ROWS_A

_MESH = plsc.VectorSubcoreMesh(core_axis_name="c", subcore_axis_name="s")
_SC_PARAMS = pltpu.CompilerParams(use_tc_tiling_on_sc=False)


def _prop_pass(grp, gpk_ref, src4_ref, dst_ref, z_ref, out_ref,
               acc, sidxA, sidxB, didxA, didxB, rowsA, rowsB,
               gsemA, gsemB, s):
    @pl.when(s < NS - 1)
    def _():
        pltpu.sync_copy(z_ref, acc.at[pl.ds(s * ROWS_A, ROWS_A)])

    @pl.when(s == NS - 1)
    def _():
        pltpu.sync_copy(z_ref.at[pl.ds(0, ROWS_B)],
                        acc.at[pl.ds((NS - 1) * ROWS_A, ROWS_B)])

    plsc.subcore_barrier()

    bb = 195 * s + jnp.minimum(s, 5)
    nblk = jnp.where(s < 5, 196, 195)
    ebase = bb * BLK

    def load_blk(jb, sidx, didx):
        pltpu.sync_copy(src4_ref.at[grp, pl.ds(ebase + jb * BLK, BLK)], sidx)
        pltpu.sync_copy(dst_ref.at[pl.ds(ebase + jb * BLK, BLK)], didx)

    def wait_gather(rows, sem):
        pltpu.make_async_copy(gpk_ref.at[pl.ds(0, BLK)], rows, sem).wait()

    load_blk(0, sidxA, didxA)
    pltpu.async_copy(gpk_ref.at[pl.ds(0, BLK)], rowsA, gsemA)

    def pair(j2, _):
        load_blk(2 * j2 + 1, sidxB, didxB)
        pltpu.async_copy(gpk_ref.at[pl.ds(0, BLK)], rowsB, gsemB)
        wait_gather(rowsA, gsemA)
        pltpu.sync_copy(rowsA, acc.at[pl.ds(0, BLK)])
        load_blk(2 * j2 + 2, sidxA, didxA)
        pltpu.async_copy(gpk_ref.at[pl.ds(0, BLK)], rowsA, gsemA)
        wait_gather(rowsB, gsemB)
        pltpu.sync_copy(rowsB, acc.at[pl.ds(0, BLK)])
        return 0

    lax.fori_loop(0, NP, pair, 0)
    wait_gather(rowsA, gsemA)
    pltpu.sync_copy(rowsA, acc.at[pl.ds(0, BLK)])

    def tblk(t, _):
        load_blk(t, sidxA, didxA)
        pltpu.async_copy(gpk_ref.at[pl.ds(0, BLK)], rowsA, gsemA).wait()
        pltpu.sync_copy(rowsA, acc.at[pl.ds(0, BLK)])
        return 0

    lax.fori_loop(2 * NP + 1, nblk, tblk, 0)

    plsc.subcore_barrier()

    obase = grp * N + s * ROWS_A

    @pl.when(s < NS - 1)
    def _():
        pltpu.sync_copy(acc.at[pl.ds(s * ROWS_A, ROWS_A)],
                        out_ref.at[pl.ds(obase, ROWS_A)])

    @pl.when(s == NS - 1)
    def _():
        pltpu.sync_copy(acc.at[pl.ds((NS - 1) * ROWS_A, ROWS_B)],
                        out_ref.at[pl.ds(obase, ROWS_B)])


def _prop_body(gpk_ref, src4_ref, dst_ref, z_ref, out_ref,
               acc, sidxA, sidxB, didxA, didxB, rowsA, rowsB,
               gsemA, gsemB):
    c = lax.axis_index("c")
    s = lax.axis_index("s")
    for p in range(2):
        _prop_pass(2 * c + p, gpk_ref, src4_ref, dst_ref, z_ref, out_ref,
                   acc, sidxA, sidxB, didxA, didxB, rowsA, rowsB,
                   gsemA, gsemB, s)


_sc_prop = pl.kernel(
    _prop_body,
    out_type=jax.ShapeDtypeStruct((4 * N, G), jnp.float32),
    mesh=_MESH,
    compiler_params=_SC_PARAMS,
    scratch_types=[
        pltpu.VMEM_SHARED((N, G), jnp.float32),
        pltpu.VMEM((BLK,), jnp.int32),
        pltpu.VMEM((BLK,), jnp.int32),
        pltpu.VMEM((BLK,), jnp.int32),
        pltpu.VMEM((BLK,), jnp.int32),
        pltpu.VMEM((BLK, G), jnp.float32),
        pltpu.VMEM((BLK, G), jnp.float32),
        pltpu.SemaphoreType.DMA,
        pltpu.SemaphoreType.DMA,
    ],
)


R = 1000
GRID = N // R
_PAD = 12


def _split_groups(g):
    z = jnp.zeros((R, _PAD), jnp.float32)
    return (g[:, 0:16],
            jnp.concatenate([g[:, 16:20], z], axis=-1),
            g[:, 20:36],
            jnp.concatenate([g[:, 36:40], z], axis=-1))


def _merge_groups(a_ref):
    return jnp.concatenate(
        [a_ref[0], a_ref[1, :, 0:4], a_ref[2], a_ref[3, :, 0:4]], axis=-1)


def _init_body(yt_ref, ys_ref, m_ref, err_ref, sig_ref, num_ref):
    i = pl.program_id(0)
    oh = (lax.broadcasted_iota(jnp.int32, (R, C), 1) == yt_ref[...]).astype(jnp.float32)
    err = jnp.where(m_ref[...] > 0, oh - ys_ref[...], 0.0)
    err_ref[...] = err

    @pl.when(i == 0)
    def _():
        sig_ref[...] = jnp.zeros_like(sig_ref)
        num_ref[...] = jnp.zeros_like(num_ref)

    sig_ref[...] += jnp.sum(jnp.abs(err))
    num_ref[...] += jnp.sum(m_ref[...])


_tc_init = pl.pallas_call(
    _init_body,
    grid=(GRID,),
    in_specs=[
        pl.BlockSpec((R, 1), lambda i: (i, 0)),
        pl.BlockSpec((R, C), lambda i: (i, 0)),
        pl.BlockSpec((R, 1), lambda i: (i, 0)),
    ],
    out_specs=[
        pl.BlockSpec((R, C), lambda i: (i, 0)),
        pl.BlockSpec((1, 1), lambda i: (0, 0)),
        pl.BlockSpec((1, 1), lambda i: (0, 0)),
    ],
    out_shape=[
        jax.ShapeDtypeStruct((N, C), jnp.float32),
        jax.ShapeDtypeStruct((1, 1), jnp.float32),
        jax.ShapeDtypeStruct((1, 1), jnp.float32),
    ],
)


def _dis_body(dg_ref, err_ref, dis_ref, g_ref):
    deg = dg_ref[:, 0:1]
    dis = jnp.where(deg > 0, lax.rsqrt(jnp.maximum(deg, 1e-12)), 0.0)
    dis_ref[...] = dis
    g0, g1, g2, g3 = _split_groups(dis * err_ref[...])
    g_ref[0] = g0
    g_ref[1] = g1
    g_ref[2] = g2
    g_ref[3] = g3


_tc_dis = pl.pallas_call(
    _dis_body,
    grid=(GRID,),
    in_specs=[
        pl.BlockSpec((R, G), lambda i: (i, 0)),
        pl.BlockSpec((R, C), lambda i: (i, 0)),
    ],
    out_specs=[
        pl.BlockSpec((R, 1), lambda i: (i, 0)),
        pl.BlockSpec((4, R, G), lambda i: (0, i, 0)),
    ],
    out_shape=[
        jax.ShapeDtypeStruct((N, 1), jnp.float32),
        jax.ShapeDtypeStruct((4, N, G), jnp.float32),
    ],
)


def _update_body(a_ref, h_ref, dis_ref, h2_ref, g_ref, *, alpha, smooth):
    a = _merge_groups(a_ref)
    dis = dis_ref[...]
    x = alpha * (dis * a) + (1.0 - alpha) * h_ref[...]
    if smooth:
        m = jnp.max(x, axis=-1, keepdims=True)
        e = jnp.exp(x - m)
        hn = e / jnp.sum(e, axis=-1, keepdims=True)
    else:
        hn = jnp.clip(x, -1.0, 1.0)
    h2_ref[...] = hn
    g0, g1, g2, g3 = _split_groups(dis * hn)
    g_ref[0] = g0
    g_ref[1] = g1
    g_ref[2] = g2
    g_ref[3] = g3


def _make_update(alpha, smooth):
    return pl.pallas_call(
        functools.partial(_update_body, alpha=alpha, smooth=smooth),
        grid=(GRID,),
        in_specs=[
            pl.BlockSpec((4, R, G), lambda i: (0, i, 0)),
            pl.BlockSpec((R, C), lambda i: (i, 0)),
            pl.BlockSpec((R, 1), lambda i: (i, 0)),
        ],
        out_specs=[
            pl.BlockSpec((R, C), lambda i: (i, 0)),
            pl.BlockSpec((4, R, G), lambda i: (0, i, 0)),
        ],
        out_shape=[
            jax.ShapeDtypeStruct((N, C), jnp.float32),
            jax.ShapeDtypeStruct((4, N, G), jnp.float32),
        ],
    )


_tc_update_clip = _make_update(ALPHA1, False)
_tc_update_soft = _make_update(ALPHA2, True)


def _scale_body(h_ref, ys_ref, yt_ref, m_ref, dis_ref, sig_ref,
                h2_ref, g_ref):
    se = h_ref[...]
    sigma = sig_ref[0, 0]
    row_abs = jnp.sum(jnp.abs(se), axis=-1, keepdims=True)
    scale = sigma / row_abs
    scale = jnp.where(jnp.isinf(scale) | (scale > 1000.0), 1.0, scale)
    yc = ys_ref[...] + scale * se
    oh = (lax.broadcasted_iota(jnp.int32, (R, C), 1) == yt_ref[...]).astype(jnp.float32)
    yin = jnp.where(m_ref[...] > 0, oh, yc)
    h2_ref[...] = yin
    g0, g1, g2, g3 = _split_groups(dis_ref[...] * yin)
    g_ref[0] = g0
    g_ref[1] = g1
    g_ref[2] = g2
    g_ref[3] = g3


_tc_scale = pl.pallas_call(
    _scale_body,
    grid=(GRID,),
    in_specs=[
        pl.BlockSpec((R, C), lambda i: (i, 0)),
        pl.BlockSpec((R, C), lambda i: (i, 0)),
        pl.BlockSpec((R, 1), lambda i: (i, 0)),
        pl.BlockSpec((R, 1), lambda i: (i, 0)),
        pl.BlockSpec((R, 1), lambda i: (i, 0)),
        pl.BlockSpec((1, 1), lambda i: (0, 0)),
    ],
    out_specs=[
        pl.BlockSpec((R, C), lambda i: (i, 0)),
        pl.BlockSpec((4, R, G), lambda i: (0, i, 0)),
    ],
    out_shape=[
        jax.ShapeDtypeStruct((N, C), jnp.float32),
        jax.ShapeDtypeStruct((4, N, G), jnp.float32),
    ],
)


def kernel(y_true, y_soft, spread_mask, eval_mask, test_mask, edge_index):
    src = edge_index[0].astype(jnp.int32)
    dst = edge_index[1].astype(jnp.int32)
    yt = y_true.astype(jnp.int32).reshape(N, 1)
    mf = spread_mask.astype(jnp.float32).reshape(N, 1)

    offs = jnp.arange(4, dtype=jnp.int32)[:, None] * N
    src4 = src[None, :] + offs
    z = jnp.zeros((ROWS_A, G), jnp.float32)
    gones = jnp.ones((4 * N, G), jnp.float32)

    err, sig_num, numel = _tc_init(yt, y_soft, mf)
    degp = _sc_prop(gones, src4, dst, z)
    dis, g = _tc_dis(degp[:N], err)

    h = err
    for _ in range(L1):
        agg = _sc_prop(g.reshape(4 * N, G), src4, dst, z)
        h, g = _tc_update_clip(agg.reshape(4, N, G), h, dis)

    sigma = (sig_num / jnp.maximum(numel, 1.0)).reshape(1, 1)
    h, g = _tc_scale(h, y_soft, yt, mf, dis, sigma)

    for _ in range(L2):
        agg = _sc_prop(g.reshape(4 * N, G), src4, dst, z)
        h, g = _tc_update_soft(agg.reshape(4, N, G), h, dis)

    return h


# speedup vs baseline: 1.6031x; 1.6031x over previous
import functools

import jax
import jax.numpy as jnp
from jax import lax
from jax.experimental import pallas as pl
from jax.experimental.pallas import tpu as pltpu
from jax.experimental.pallas import tpu_sc as plsc

N = 100000
C = 40
G = 16
E = 1600000
L1 = 10
ALPHA1 = 0.8
L2 = 10
ALPHA2 = 0.8

NC = 2
NS = 16
CH = 128

BLK = 800
NBLOCK = E // BLK
NBT = NBLOCK // NS
NP = (NBT - 1) // 2

ROWS_A = 6256
ROWS_B = N - 15 * ROWS_A

_MESH = plsc.VectorSubcoreMesh(core_axis_name="c", subcore_axis_name="s")
_SC_PARAMS = pltpu.CompilerParams(use_tc_tiling_on_sc=False)


def _prop_pass(grp, gpk_ref, sd_ref, z_ref, out_ref,
               acc, idxA, idxB, rowsA, rowsB,
               gsemA, gsemB, s):
    @pl.when(s < NS - 1)
    def _():
        pltpu.sync_copy(z_ref, acc.at[pl.ds(s * ROWS_A, ROWS_A)])

    @pl.when(s == NS - 1)
    def _():
        pltpu.sync_copy(z_ref.at[pl.ds(0, ROWS_B)],
                        acc.at[pl.ds((NS - 1) * ROWS_A, ROWS_B)])

    plsc.subcore_barrier()

    bb = s * NBT

    def load_blk(jb, idx2):
        pltpu.sync_copy(sd_ref.at[grp, pl.ds(2 * (bb + jb), 2)], idx2)

    def wait_gather(rows, sem):
        pltpu.make_async_copy(gpk_ref.at[pl.ds(0, BLK)], rows, sem).wait()

    load_blk(0, idxA)
    pltpu.async_copy(gpk_ref.at[idxA.at[0]], rowsA, gsemA)

    def pair(j2, _):
        load_blk(2 * j2 + 1, idxB)
        pltpu.async_copy(gpk_ref.at[idxB.at[0]], rowsB, gsemB)
        wait_gather(rowsA, gsemA)
        pltpu.sync_copy(rowsA, acc.at[idxA.at[1]], add=True)
        load_blk(2 * j2 + 2, idxA)
        pltpu.async_copy(gpk_ref.at[idxA.at[0]], rowsA, gsemA)
        wait_gather(rowsB, gsemB)
        pltpu.sync_copy(rowsB, acc.at[idxB.at[1]], add=True)
        return 0

    lax.fori_loop(0, NP, pair, 0)
    wait_gather(rowsA, gsemA)
    pltpu.sync_copy(rowsA, acc.at[idxA.at[1]], add=True)

    plsc.subcore_barrier()

    obase = grp * N + s * ROWS_A

    @pl.when(s < NS - 1)
    def _():
        pltpu.sync_copy(acc.at[pl.ds(s * ROWS_A, ROWS_A)],
                        out_ref.at[pl.ds(obase, ROWS_A)])

    @pl.when(s == NS - 1)
    def _():
        pltpu.sync_copy(acc.at[pl.ds((NS - 1) * ROWS_A, ROWS_B)],
                        out_ref.at[pl.ds(obase, ROWS_B)])


def _prop_body(gpk_ref, sd_ref, z_ref, out_ref,
               acc, idxA, idxB, rowsA, rowsB,
               gsemA, gsemB):
    c = lax.axis_index("c")
    s = lax.axis_index("s")
    for p in range(2):
        _prop_pass(2 * c + p, gpk_ref, sd_ref, z_ref, out_ref,
                   acc, idxA, idxB, rowsA, rowsB,
                   gsemA, gsemB, s)


_sc_prop = pl.kernel(
    _prop_body,
    out_type=jax.ShapeDtypeStruct((4 * N, G), jnp.float32),
    mesh=_MESH,
    compiler_params=_SC_PARAMS,
    scratch_types=[
        pltpu.VMEM_SHARED((N, G), jnp.float32),
        pltpu.VMEM((2, BLK), jnp.int32),
        pltpu.VMEM((2, BLK), jnp.int32),
        pltpu.VMEM((BLK, G), jnp.float32),
        pltpu.VMEM((BLK, G), jnp.float32),
        pltpu.SemaphoreType.DMA,
        pltpu.SemaphoreType.DMA,
    ],
)


R = 1000
GRID = N // R
_PAD = 12


def _split_groups(g):
    z = jnp.zeros((R, _PAD), jnp.float32)
    return (g[:, 0:16],
            jnp.concatenate([g[:, 16:20], z], axis=-1),
            g[:, 20:36],
            jnp.concatenate([g[:, 36:40], z], axis=-1))


def _merge_groups(a_ref):
    return jnp.concatenate(
        [a_ref[0], a_ref[1, :, 0:4], a_ref[2], a_ref[3, :, 0:4]], axis=-1)


def _init_body(yt_ref, ys_ref, m_ref, err_ref, sig_ref, num_ref):
    i = pl.program_id(0)
    oh = (lax.broadcasted_iota(jnp.int32, (R, C), 1) == yt_ref[...]).astype(jnp.float32)
    err = jnp.where(m_ref[...] > 0, oh - ys_ref[...], 0.0)
    err_ref[...] = err

    @pl.when(i == 0)
    def _():
        sig_ref[...] = jnp.zeros_like(sig_ref)
        num_ref[...] = jnp.zeros_like(num_ref)

    sig_ref[...] += jnp.sum(jnp.abs(err))
    num_ref[...] += jnp.sum(m_ref[...])


_tc_init = pl.pallas_call(
    _init_body,
    grid=(GRID,),
    in_specs=[
        pl.BlockSpec((R, 1), lambda i: (i, 0)),
        pl.BlockSpec((R, C), lambda i: (i, 0)),
        pl.BlockSpec((R, 1), lambda i: (i, 0)),
    ],
    out_specs=[
        pl.BlockSpec((R, C), lambda i: (i, 0)),
        pl.BlockSpec((1, 1), lambda i: (0, 0)),
        pl.BlockSpec((1, 1), lambda i: (0, 0)),
    ],
    out_shape=[
        jax.ShapeDtypeStruct((N, C), jnp.float32),
        jax.ShapeDtypeStruct((1, 1), jnp.float32),
        jax.ShapeDtypeStruct((1, 1), jnp.float32),
    ],
)


def _dis_body(dg_ref, err_ref, dis_ref, g_ref):
    deg = dg_ref[:, 0:1]
    dis = jnp.where(deg > 0, lax.rsqrt(jnp.maximum(deg, 1e-12)), 0.0)
    dis_ref[...] = dis
    g0, g1, g2, g3 = _split_groups(dis * err_ref[...])
    g_ref[0] = g0
    g_ref[1] = g1
    g_ref[2] = g2
    g_ref[3] = g3


_tc_dis = pl.pallas_call(
    _dis_body,
    grid=(GRID,),
    in_specs=[
        pl.BlockSpec((R, G), lambda i: (i, 0)),
        pl.BlockSpec((R, C), lambda i: (i, 0)),
    ],
    out_specs=[
        pl.BlockSpec((R, 1), lambda i: (i, 0)),
        pl.BlockSpec((4, R, G), lambda i: (0, i, 0)),
    ],
    out_shape=[
        jax.ShapeDtypeStruct((N, 1), jnp.float32),
        jax.ShapeDtypeStruct((4, N, G), jnp.float32),
    ],
)


def _update_body(a_ref, h_ref, dis_ref, h2_ref, g_ref, *, alpha, smooth):
    a = _merge_groups(a_ref)
    dis = dis_ref[...]
    x = alpha * (dis * a) + (1.0 - alpha) * h_ref[...]
    if smooth:
        m = jnp.max(x, axis=-1, keepdims=True)
        e = jnp.exp(x - m)
        hn = e / jnp.sum(e, axis=-1, keepdims=True)
    else:
        hn = jnp.clip(x, -1.0, 1.0)
    h2_ref[...] = hn
    g0, g1, g2, g3 = _split_groups(dis * hn)
    g_ref[0] = g0
    g_ref[1] = g1
    g_ref[2] = g2
    g_ref[3] = g3


def _make_update(alpha, smooth):
    return pl.pallas_call(
        functools.partial(_update_body, alpha=alpha, smooth=smooth),
        grid=(GRID,),
        in_specs=[
            pl.BlockSpec((4, R, G), lambda i: (0, i, 0)),
            pl.BlockSpec((R, C), lambda i: (i, 0)),
            pl.BlockSpec((R, 1), lambda i: (i, 0)),
        ],
        out_specs=[
            pl.BlockSpec((R, C), lambda i: (i, 0)),
            pl.BlockSpec((4, R, G), lambda i: (0, i, 0)),
        ],
        out_shape=[
            jax.ShapeDtypeStruct((N, C), jnp.float32),
            jax.ShapeDtypeStruct((4, N, G), jnp.float32),
        ],
    )


_tc_update_clip = _make_update(ALPHA1, False)
_tc_update_soft = _make_update(ALPHA2, True)


def _scale_body(h_ref, ys_ref, yt_ref, m_ref, dis_ref, sig_ref,
                h2_ref, g_ref):
    se = h_ref[...]
    sigma = sig_ref[0, 0]
    row_abs = jnp.sum(jnp.abs(se), axis=-1, keepdims=True)
    scale = sigma / row_abs
    scale = jnp.where(jnp.isinf(scale) | (scale > 1000.0), 1.0, scale)
    yc = ys_ref[...] + scale * se
    oh = (lax.broadcasted_iota(jnp.int32, (R, C), 1) == yt_ref[...]).astype(jnp.float32)
    yin = jnp.where(m_ref[...] > 0, oh, yc)
    h2_ref[...] = yin
    g0, g1, g2, g3 = _split_groups(dis_ref[...] * yin)
    g_ref[0] = g0
    g_ref[1] = g1
    g_ref[2] = g2
    g_ref[3] = g3


_tc_scale = pl.pallas_call(
    _scale_body,
    grid=(GRID,),
    in_specs=[
        pl.BlockSpec((R, C), lambda i: (i, 0)),
        pl.BlockSpec((R, C), lambda i: (i, 0)),
        pl.BlockSpec((R, 1), lambda i: (i, 0)),
        pl.BlockSpec((R, 1), lambda i: (i, 0)),
        pl.BlockSpec((R, 1), lambda i: (i, 0)),
        pl.BlockSpec((1, 1), lambda i: (0, 0)),
    ],
    out_specs=[
        pl.BlockSpec((R, C), lambda i: (i, 0)),
        pl.BlockSpec((4, R, G), lambda i: (0, i, 0)),
    ],
    out_shape=[
        jax.ShapeDtypeStruct((N, C), jnp.float32),
        jax.ShapeDtypeStruct((4, N, G), jnp.float32),
    ],
)


def kernel(y_true, y_soft, spread_mask, eval_mask, test_mask, edge_index):
    src = edge_index[0].astype(jnp.int32)
    dst = edge_index[1].astype(jnp.int32)
    yt = y_true.astype(jnp.int32).reshape(N, 1)
    mf = spread_mask.astype(jnp.float32).reshape(N, 1)

    dst_b = dst.reshape(NBLOCK, BLK)
    sd = jnp.stack([
        jnp.stack([(src + g * N).reshape(NBLOCK, BLK), dst_b],
                  axis=1).reshape(2 * NBLOCK, BLK)
        for g in range(4)])
    z = jnp.zeros((ROWS_A, G), jnp.float32)
    gones = jnp.ones((4 * N, G), jnp.float32)

    err, sig_num, numel = _tc_init(yt, y_soft, mf)
    degp = _sc_prop(gones, sd, z)
    dis, g = _tc_dis(degp[:N], err)

    h = err
    for _ in range(L1):
        agg = _sc_prop(g.reshape(4 * N, G), sd, z)
        h, g = _tc_update_clip(agg.reshape(4, N, G), h, dis)

    sigma = (sig_num / jnp.maximum(numel, 1.0)).reshape(1, 1)
    h, g = _tc_scale(h, y_soft, yt, mf, dis, sigma)

    for _ in range(L2):
        agg = _sc_prop(g.reshape(4 * N, G), sd, z)
        h, g = _tc_update_soft(agg.reshape(4, N, G), h, dis)

    return h
